# baseline (speedup 1.0000x reference)
import jax
import jax.numpy as jnp
from jax.experimental import pallas as pl
from jax.experimental.pallas import tpu as pltpu


def _fts_kernel(seq_ref, wt_ref, fts_ref):
    fts = jnp.dot(seq_ref[...], wt_ref[...],
                  preferred_element_type=jnp.float32,
                  precision=jax.lax.Precision.HIGHEST)
    fts_ref[...] = fts.astype(jnp.bfloat16)


def _agg_kernel(fts_ref, b_ref, adj0_ref, adj1_ref, out_ref):
    bm = adj0_ref.shape[0]
    acc0 = jnp.dot(adj0_ref[...].astype(jnp.bfloat16), fts_ref[...],
                   preferred_element_type=jnp.float32)
    out_ref[:bm, :] = acc0 + b_ref[...]
    acc1 = jnp.dot(adj1_ref[...].astype(jnp.bfloat16), fts_ref[...],
                   preferred_element_type=jnp.float32)
    out_ref[bm:, :] = acc1 + b_ref[...]


def kernel(seq, adj, W, b):
    batch, n, in_ft = seq.shape
    out_ft = W.shape[0]
    seq2 = seq.reshape(batch * n, in_ft)
    adj2 = adj.reshape(batch * n, n)
    wt = W.T
    b2 = b.reshape(1, out_ft)

    fm = 2000
    fts = pl.pallas_call(
        _fts_kernel,
        grid=(n // fm,),
        in_specs=[
            pl.BlockSpec((fm, in_ft), lambda i: (i, 0)),
            pl.BlockSpec((in_ft, out_ft), lambda i: (0, 0)),
        ],
        out_specs=pl.BlockSpec((fm, out_ft), lambda i: (i, 0)),
        out_shape=jax.ShapeDtypeStruct((n, out_ft), jnp.bfloat16),
        compiler_params=pltpu.CompilerParams(
            dimension_semantics=("arbitrary",),
        ),
    )(seq2, wt)

    bm = 200
    out = pl.pallas_call(
        _agg_kernel,
        grid=(n // (2 * bm),),
        in_specs=[
            pl.BlockSpec((n, out_ft), lambda i: (0, 0)),
            pl.BlockSpec((1, out_ft), lambda i: (0, 0)),
            pl.BlockSpec((bm, n), lambda i: (2 * i, 0)),
            pl.BlockSpec((bm, n), lambda i: (2 * i + 1, 0)),
        ],
        out_specs=pl.BlockSpec((2 * bm, out_ft), lambda i: (i, 0)),
        out_shape=jax.ShapeDtypeStruct((n, out_ft), jnp.float32),
        compiler_params=pltpu.CompilerParams(
            dimension_semantics=("arbitrary",),
        ),
    )(fts, b2, adj2, adj2)

    return out.reshape(batch, n, out_ft)


# speedup vs baseline: 1.0553x; 1.0553x over previous
import jax
import jax.numpy as jnp
from jax.experimental import pallas as pl
from jax.experimental.pallas import tpu as pltpu

_NBUF = 4
_BM = 200


def _gcn_kernel(seq_ref, wt_ref, b_ref, adj_ref, out_ref,
                fts_ref, buf_ref, sem_ref):
    n = seq_ref.shape[0]
    nsteps = n // _BM

    def copy(k):
        return pltpu.make_async_copy(
            adj_ref.at[pl.ds(k * _BM, _BM), :],
            buf_ref.at[k % _NBUF],
            sem_ref.at[k % _NBUF],
        )

    for k in range(_NBUF - 1):
        copy(k).start()

    fts = jnp.dot(seq_ref[...], wt_ref[...],
                  preferred_element_type=jnp.float32,
                  precision=jax.lax.Precision.HIGHEST)
    fts_ref[...] = fts.astype(jnp.bfloat16)

    bias = b_ref[...]
    for k in range(nsteps):
        copy(k).wait()
        if k + _NBUF - 1 < nsteps:
            copy(k + _NBUF - 1).start()
        acc = jnp.dot(buf_ref[k % _NBUF].astype(jnp.bfloat16), fts_ref[...],
                      preferred_element_type=jnp.float32)
        out_ref[pl.ds(k * _BM, _BM), :] = acc + bias


def kernel(seq, adj, W, b):
    batch, n, in_ft = seq.shape
    out_ft = W.shape[0]
    seq2 = seq.reshape(batch * n, in_ft)
    adj2 = adj.reshape(batch * n, n)
    wt = W.T
    b2 = b.reshape(1, out_ft)

    out = pl.pallas_call(
        _gcn_kernel,
        in_specs=[
            pl.BlockSpec((n, in_ft), lambda: (0, 0)),
            pl.BlockSpec((in_ft, out_ft), lambda: (0, 0)),
            pl.BlockSpec((1, out_ft), lambda: (0, 0)),
            pl.BlockSpec(memory_space=pl.ANY),
        ],
        out_specs=pl.BlockSpec((n, out_ft), lambda: (0, 0)),
        out_shape=jax.ShapeDtypeStruct((n, out_ft), jnp.float32),
        scratch_shapes=[
            pltpu.VMEM((n, out_ft), jnp.bfloat16),
            pltpu.VMEM((_NBUF, _BM, n), jnp.float32),
            pltpu.SemaphoreType.DMA((_NBUF,)),
        ],
    )(seq2, wt, b2, adj2)

    return out.reshape(batch, n, out_ft)
